# Initial kernel scaffold; baseline (speedup 1.0000x reference)
import jax
import jax.numpy as jnp
from jax.experimental import pallas as pl


def kernel(x, edge_index, W1_0, W1_1, b1, W2_0, W2_1, b2):
    raise NotImplementedError("write your pallas kernel here")


# speedup vs baseline: 30.6295x; 30.6295x over previous
import functools

import jax
import jax.numpy as jnp
from jax import lax
from jax.experimental import pallas as pl
from jax.experimental.pallas import tpu as pltpu
from jax.experimental.pallas import tpu_sc as plsc

N_NODES = 10000
N_EDGES = 320000
D_FEAT = 128
D_HID = 16

NC = 2
NS = 16
NW = NC * NS
EPT = N_EDGES // NW
CHUNK = 125
NCHUNK = EPT // CHUNK
TBL = 10240
ZROWS = TBL // NS
ZSTEP = 128

@functools.cache
def _mesh():
    return plsc.VectorSubcoreMesh(
        core_axis_name="c", subcore_axis_name="s", num_cores=NC, num_subcores=NS
    )


def _zero_stripe(acc, zbuf, s):
    def zrow(i, _):
        zbuf[i, :] = jnp.zeros((16,), jnp.float32)
        return 0

    lax.fori_loop(0, ZSTEP, zrow, 0)

    def zcp(i, _):
        pltpu.sync_copy(zbuf, acc.at[pl.ds(s * ZROWS + i * ZSTEP, ZSTEP)])
        return 0

    lax.fori_loop(0, ZROWS // ZSTEP, zcp, 0)


def _copy_out(acc, out_hbm, c, s):
    pltpu.sync_copy(
        acc.at[pl.ds(s * ZROWS, ZROWS)], out_hbm.at[c, pl.ds(s * ZROWS, ZROWS)]
    )


@functools.cache
def _sc_degree_kernel():
    return pl.kernel(
        _sc_degree_body,
        out_type=jax.ShapeDtypeStruct((NC, TBL, D_HID), jnp.float32),
        mesh=_mesh(),
        compiler_params=pltpu.CompilerParams(use_tc_tiling_on_sc=False),
        scratch_types=[
            pltpu.VMEM_SHARED((TBL, D_HID), jnp.float32),
            pltpu.VMEM((NCHUNK, CHUNK), jnp.int32),
            pltpu.VMEM((CHUNK, D_HID), jnp.float32),
            pltpu.VMEM((ZSTEP, D_HID), jnp.float32),
        ],
    )


def _sc_degree_body(col_hbm, out_hbm, acc, cidx, ones_b, zbuf):
    c = lax.axis_index("c")
    s = lax.axis_index("s")
    wid = c * NS + s
    _zero_stripe(acc, zbuf, s)

    def orow(i, _):
        ones_b[i, :] = jnp.ones((16,), jnp.float32)
        return 0

    lax.fori_loop(0, CHUNK, orow, 0)
    pltpu.sync_copy(col_hbm.at[wid], cidx)
    plsc.subcore_barrier()

    def step(j, _):
        pltpu.sync_copy(ones_b, acc.at[cidx.at[j]], add=True)
        return 0

    lax.fori_loop(0, NCHUNK, step, 0)
    plsc.subcore_barrier()
    _copy_out(acc, out_hbm, c, s)


@functools.cache
def _sc_propagate_kernel():
    return pl.kernel(
        _sc_propagate_body,
        out_type=jax.ShapeDtypeStruct((NC, TBL, D_HID), jnp.float32),
        mesh=_mesh(),
        compiler_params=pltpu.CompilerParams(use_tc_tiling_on_sc=False),
        scratch_types=[
            pltpu.VMEM_SHARED((TBL, D_HID), jnp.float32),
            pltpu.VMEM((NCHUNK, CHUNK), jnp.int32),
            pltpu.VMEM((NCHUNK, CHUNK), jnp.int32),
            pltpu.VMEM((CHUNK, D_HID), jnp.float32),
            pltpu.VMEM((ZSTEP, D_HID), jnp.float32),
        ],
    )


def _sc_propagate_body(row_hbm, col_hbm, tbl_hbm, out_hbm, acc, ridx, cidx, buf, zbuf):
    c = lax.axis_index("c")
    s = lax.axis_index("s")
    wid = c * NS + s
    _zero_stripe(acc, zbuf, s)
    pltpu.sync_copy(row_hbm.at[wid], ridx)
    pltpu.sync_copy(col_hbm.at[wid], cidx)
    plsc.subcore_barrier()

    def step(j, _):
        pltpu.sync_copy(tbl_hbm.at[ridx.at[j]], buf)
        pltpu.sync_copy(buf, acc.at[cidx.at[j]], add=True)
        return 0

    lax.fori_loop(0, NCHUNK, step, 0)
    plsc.subcore_barrier()
    _copy_out(acc, out_hbm, c, s)


ROWS_B = 2000
GRID = N_NODES // ROWS_B


def _tc_prep_body(x_ref, w_ref, deg_ref, a0_ref, ys_ref, dis_ref):
    deg = deg_ref[0] + deg_ref[1]
    dis = jnp.where(deg > 0, lax.rsqrt(deg), 0.0)
    xw = jnp.dot(x_ref[...], w_ref[...], preferred_element_type=jnp.float32)
    a0_ref[...] = xw[:, :D_HID]
    ys_ref[...] = dis * xw[:, D_HID:]
    dis_ref[...] = dis


def _tc_prep(x, w1c, degp):
    return pl.pallas_call(
        _tc_prep_body,
        grid=(GRID,),
        in_specs=[
            pl.BlockSpec((ROWS_B, D_FEAT), lambda i: (i, 0)),
            pl.BlockSpec((D_FEAT, 2 * D_HID), lambda i: (0, 0)),
            pl.BlockSpec((NC, ROWS_B, D_HID), lambda i: (0, i, 0)),
        ],
        out_specs=[pl.BlockSpec((ROWS_B, D_HID), lambda i: (i, 0))] * 3,
        out_shape=[jax.ShapeDtypeStruct((N_NODES, D_HID), jnp.float32)] * 3,
    )(x, w1c, degp)


def _tc_mid_body(a0_ref, pp_ref, dis_ref, w_ref, b1_ref, c0_ref, zs_ref):
    dis = dis_ref[...]
    h = jnp.maximum(a0_ref[...] + dis * (pp_ref[0] + pp_ref[1]) + b1_ref[...], 0.0)
    hw = jnp.dot(h, w_ref[...], preferred_element_type=jnp.float32)
    c0_ref[...] = hw[:, :D_HID]
    zs_ref[...] = dis * hw[:, D_HID:]


def _tc_mid(a0, pparts, dis, w2c, b1):
    return pl.pallas_call(
        _tc_mid_body,
        grid=(GRID,),
        in_specs=[
            pl.BlockSpec((ROWS_B, D_HID), lambda i: (i, 0)),
            pl.BlockSpec((NC, ROWS_B, D_HID), lambda i: (0, i, 0)),
            pl.BlockSpec((ROWS_B, D_HID), lambda i: (i, 0)),
            pl.BlockSpec((D_HID, 2 * D_HID), lambda i: (0, 0)),
            pl.BlockSpec((1, D_HID), lambda i: (0, 0)),
        ],
        out_specs=[pl.BlockSpec((ROWS_B, D_HID), lambda i: (i, 0))] * 2,
        out_shape=[jax.ShapeDtypeStruct((N_NODES, D_HID), jnp.float32)] * 2,
    )(a0, pparts, dis, w2c, b1)


def _tc_final_body(c0_ref, qp_ref, dis_ref, b2_ref, out_ref):
    o = c0_ref[...] + dis_ref[...] * (qp_ref[0] + qp_ref[1]) + b2_ref[...]
    m = jnp.max(o, axis=1, keepdims=True)
    e = jnp.exp(o - m)
    out_ref[...] = (o - m) - jnp.log(jnp.sum(e, axis=1, keepdims=True))


def _tc_final(c0, qparts, dis, b2):
    return pl.pallas_call(
        _tc_final_body,
        grid=(GRID,),
        in_specs=[
            pl.BlockSpec((ROWS_B, D_HID), lambda i: (i, 0)),
            pl.BlockSpec((NC, ROWS_B, D_HID), lambda i: (0, i, 0)),
            pl.BlockSpec((ROWS_B, D_HID), lambda i: (i, 0)),
            pl.BlockSpec((1, D_HID), lambda i: (0, 0)),
        ],
        out_specs=pl.BlockSpec((ROWS_B, D_HID), lambda i: (i, 0)),
        out_shape=jax.ShapeDtypeStruct((N_NODES, D_HID), jnp.float32),
    )(c0, qparts, dis, b2)


def kernel(x, edge_index, W1_0, W1_1, b1, W2_0, W2_1, b2):
    ei = edge_index.astype(jnp.int32)
    row3 = ei[0].reshape(NW, NCHUNK, CHUNK)
    col3 = ei[1].reshape(NW, NCHUNK, CHUNK)
    degp = _sc_degree_kernel()(col3)
    w1c = jnp.concatenate([W1_0, W1_1], axis=1)
    a0, ys, dis = _tc_prep(x, w1c, degp)
    pparts = _sc_propagate_kernel()(row3, col3, ys)
    w2c = jnp.concatenate([W2_0, W2_1], axis=1)
    c0, zs = _tc_mid(a0, pparts, dis, w2c, b1.reshape(1, D_HID))
    qparts = _sc_propagate_kernel()(row3, col3, zs)
    return _tc_final(c0, qparts, dis, b2.reshape(1, D_HID))


# speedup vs baseline: 35.7431x; 1.1670x over previous
import functools

import jax
import jax.numpy as jnp
from jax import lax
from jax.experimental import pallas as pl
from jax.experimental.pallas import tpu as pltpu
from jax.experimental.pallas import tpu_sc as plsc

N_NODES = 10000
N_EDGES = 320000
D_FEAT = 128
D_HID = 16

NC = 2
NS = 16
NW = NC * NS
EPT = N_EDGES // NW
CHUNK = 125
NCHUNK = EPT // CHUNK
TBL = 10240
ZROWS = TBL // NS
ZSTEP = 128

@functools.cache
def _mesh():
    return plsc.VectorSubcoreMesh(
        core_axis_name="c", subcore_axis_name="s", num_cores=NC, num_subcores=NS
    )


def _zero_stripe(acc, zbuf, s):
    def zrow(i, _):
        zbuf[i, :] = jnp.zeros((16,), jnp.float32)
        return 0

    lax.fori_loop(0, ZSTEP, zrow, 0)

    def zcp(i, _):
        pltpu.sync_copy(zbuf, acc.at[pl.ds(s * ZROWS + i * ZSTEP, ZSTEP)])
        return 0

    lax.fori_loop(0, ZROWS // ZSTEP, zcp, 0)


def _copy_out(acc, out_hbm, c, s):
    pltpu.sync_copy(
        acc.at[pl.ds(s * ZROWS, ZROWS)], out_hbm.at[c, pl.ds(s * ZROWS, ZROWS)]
    )


@functools.cache
def _sc_degree_kernel():
    return pl.kernel(
        _sc_degree_body,
        out_type=jax.ShapeDtypeStruct((NC, TBL, D_HID), jnp.float32),
        mesh=_mesh(),
        compiler_params=pltpu.CompilerParams(use_tc_tiling_on_sc=False),
        scratch_types=[
            pltpu.VMEM_SHARED((TBL, D_HID), jnp.float32),
            pltpu.VMEM((NCHUNK, CHUNK), jnp.int32),
            pltpu.VMEM((CHUNK, D_HID), jnp.float32),
            pltpu.VMEM((ZSTEP, D_HID), jnp.float32),
            pltpu.SemaphoreType.DMA,
        ],
    )


DEG_WIN = 8


def _sc_degree_body(col_hbm, out_hbm, acc, cidx, ones_b, zbuf, ssem):
    c = lax.axis_index("c")
    s = lax.axis_index("s")
    wid = c * NS + s
    _zero_stripe(acc, zbuf, s)

    def orow(i, _):
        ones_b[i, :] = jnp.ones((16,), jnp.float32)
        return 0

    lax.fori_loop(0, CHUNK, orow, 0)
    pltpu.sync_copy(col_hbm.at[wid], cidx)
    plsc.subcore_barrier()

    def step(j, _):
        @pl.when(j >= DEG_WIN)
        def _():
            pltpu.make_async_copy(ones_b, acc.at[cidx.at[j - DEG_WIN]], ssem).wait()

        pltpu.async_copy(ones_b, acc.at[cidx.at[j]], ssem, add=True)
        return 0

    lax.fori_loop(0, NCHUNK, step, 0)

    def drain(j, _):
        pltpu.make_async_copy(ones_b, acc.at[cidx.at[NCHUNK - DEG_WIN + j]], ssem).wait()
        return 0

    lax.fori_loop(0, DEG_WIN, drain, 0)
    plsc.subcore_barrier()
    _copy_out(acc, out_hbm, c, s)


@functools.cache
def _sc_propagate_kernel():
    return pl.kernel(
        _sc_propagate_body,
        out_type=jax.ShapeDtypeStruct((NC, TBL, D_HID), jnp.float32),
        mesh=_mesh(),
        compiler_params=pltpu.CompilerParams(use_tc_tiling_on_sc=False),
        scratch_types=[
            pltpu.VMEM_SHARED((TBL, D_HID), jnp.float32),
            pltpu.VMEM((NCHUNK, CHUNK), jnp.int32),
            pltpu.VMEM((NCHUNK, CHUNK), jnp.int32),
            pltpu.VMEM((CHUNK, D_HID), jnp.float32),
            pltpu.VMEM((CHUNK, D_HID), jnp.float32),
            pltpu.VMEM((CHUNK, D_HID), jnp.float32),
            pltpu.VMEM((CHUNK, D_HID), jnp.float32),
            pltpu.VMEM((ZSTEP, D_HID), jnp.float32),
            pltpu.SemaphoreType.DMA,
            pltpu.SemaphoreType.DMA,
        ],
    )


def _sc_propagate_body(
    row_hbm, col_hbm, tbl_hbm, out_hbm, acc, ridx, cidx, b0, b1, b2, b3, zbuf,
    gsem, ssem
):
    c = lax.axis_index("c")
    s = lax.axis_index("s")
    wid = c * NS + s
    bufs = (b0, b1, b2, b3)
    _zero_stripe(acc, zbuf, s)
    pltpu.sync_copy(row_hbm.at[wid], ridx)
    pltpu.sync_copy(col_hbm.at[wid], cidx)
    plsc.subcore_barrier()

    pltpu.async_copy(tbl_hbm.at[ridx.at[0]], bufs[0], gsem)
    pltpu.async_copy(tbl_hbm.at[ridx.at[1]], bufs[1], gsem)

    def outer(i, _):
        for b4 in range(4):
            j = 4 * i + b4
            pltpu.make_async_copy(tbl_hbm.at[ridx.at[j]], bufs[b4], gsem).wait()
            pltpu.async_copy(bufs[b4], acc.at[cidx.at[j]], ssem, add=True)

            @pl.when(j >= 2)
            def _():
                pltpu.make_async_copy(
                    bufs[(b4 + 2) % 4], acc.at[cidx.at[j - 2]], ssem
                ).wait()

            @pl.when(j + 2 < NCHUNK)
            def _():
                pltpu.async_copy(
                    tbl_hbm.at[ridx.at[j + 2]], bufs[(b4 + 2) % 4], gsem
                )
        return 0

    lax.fori_loop(0, NCHUNK // 4, outer, 0)
    pltpu.make_async_copy(bufs[2], acc.at[cidx.at[NCHUNK - 2]], ssem).wait()
    pltpu.make_async_copy(bufs[3], acc.at[cidx.at[NCHUNK - 1]], ssem).wait()
    plsc.subcore_barrier()
    _copy_out(acc, out_hbm, c, s)


ROWS_B = 2000
GRID = N_NODES // ROWS_B


def _tc_prep_body(x_ref, w_ref, deg_ref, a0_ref, ys_ref, dis_ref):
    deg = deg_ref[0] + deg_ref[1]
    dis = jnp.where(deg > 0, lax.rsqrt(deg), 0.0)
    xw = jnp.dot(x_ref[...], w_ref[...], preferred_element_type=jnp.float32)
    a0_ref[...] = xw[:, :D_HID]
    ys_ref[...] = dis * xw[:, D_HID:]
    dis_ref[...] = dis


def _tc_prep(x, w1c, degp):
    return pl.pallas_call(
        _tc_prep_body,
        grid=(GRID,),
        in_specs=[
            pl.BlockSpec((ROWS_B, D_FEAT), lambda i: (i, 0)),
            pl.BlockSpec((D_FEAT, 2 * D_HID), lambda i: (0, 0)),
            pl.BlockSpec((NC, ROWS_B, D_HID), lambda i: (0, i, 0)),
        ],
        out_specs=[pl.BlockSpec((ROWS_B, D_HID), lambda i: (i, 0))] * 3,
        out_shape=[jax.ShapeDtypeStruct((N_NODES, D_HID), jnp.float32)] * 3,
    )(x, w1c, degp)


def _tc_mid_body(a0_ref, pp_ref, dis_ref, w_ref, b1_ref, c0_ref, zs_ref):
    dis = dis_ref[...]
    h = jnp.maximum(a0_ref[...] + dis * (pp_ref[0] + pp_ref[1]) + b1_ref[...], 0.0)
    hw = jnp.dot(h, w_ref[...], preferred_element_type=jnp.float32)
    c0_ref[...] = hw[:, :D_HID]
    zs_ref[...] = dis * hw[:, D_HID:]


def _tc_mid(a0, pparts, dis, w2c, b1):
    return pl.pallas_call(
        _tc_mid_body,
        grid=(GRID,),
        in_specs=[
            pl.BlockSpec((ROWS_B, D_HID), lambda i: (i, 0)),
            pl.BlockSpec((NC, ROWS_B, D_HID), lambda i: (0, i, 0)),
            pl.BlockSpec((ROWS_B, D_HID), lambda i: (i, 0)),
            pl.BlockSpec((D_HID, 2 * D_HID), lambda i: (0, 0)),
            pl.BlockSpec((1, D_HID), lambda i: (0, 0)),
        ],
        out_specs=[pl.BlockSpec((ROWS_B, D_HID), lambda i: (i, 0))] * 2,
        out_shape=[jax.ShapeDtypeStruct((N_NODES, D_HID), jnp.float32)] * 2,
    )(a0, pparts, dis, w2c, b1)


def _tc_final_body(c0_ref, qp_ref, dis_ref, b2_ref, out_ref):
    o = c0_ref[...] + dis_ref[...] * (qp_ref[0] + qp_ref[1]) + b2_ref[...]
    m = jnp.max(o, axis=1, keepdims=True)
    e = jnp.exp(o - m)
    out_ref[...] = (o - m) - jnp.log(jnp.sum(e, axis=1, keepdims=True))


def _tc_final(c0, qparts, dis, b2):
    return pl.pallas_call(
        _tc_final_body,
        grid=(GRID,),
        in_specs=[
            pl.BlockSpec((ROWS_B, D_HID), lambda i: (i, 0)),
            pl.BlockSpec((NC, ROWS_B, D_HID), lambda i: (0, i, 0)),
            pl.BlockSpec((ROWS_B, D_HID), lambda i: (i, 0)),
            pl.BlockSpec((1, D_HID), lambda i: (0, 0)),
        ],
        out_specs=pl.BlockSpec((ROWS_B, D_HID), lambda i: (i, 0)),
        out_shape=jax.ShapeDtypeStruct((N_NODES, D_HID), jnp.float32),
    )(c0, qparts, dis, b2)


def kernel(x, edge_index, W1_0, W1_1, b1, W2_0, W2_1, b2):
    ei = edge_index.astype(jnp.int32)
    row3 = ei[0].reshape(NW, NCHUNK, CHUNK)
    col3 = ei[1].reshape(NW, NCHUNK, CHUNK)
    degp = _sc_degree_kernel()(col3)
    w1c = jnp.concatenate([W1_0, W1_1], axis=1)
    a0, ys, dis = _tc_prep(x, w1c, degp)
    pparts = _sc_propagate_kernel()(row3, col3, ys)
    w2c = jnp.concatenate([W2_0, W2_1], axis=1)
    c0, zs = _tc_mid(a0, pparts, dis, w2c, b1.reshape(1, D_HID))
    qparts = _sc_propagate_kernel()(row3, col3, zs)
    return _tc_final(c0, qparts, dis, b2.reshape(1, D_HID))


# speedup vs baseline: 45.2805x; 1.2668x over previous
import functools

import jax
import jax.numpy as jnp
from jax import lax
from jax.experimental import pallas as pl
from jax.experimental.pallas import tpu as pltpu
from jax.experimental.pallas import tpu_sc as plsc

N_NODES = 10000
N_EDGES = 320000
D_FEAT = 128
D_HID = 16

NC = 2
NS = 16
NW = NC * NS
EPT = N_EDGES // NW
CHUNK = 125
NCHUNK = EPT // CHUNK
TBL = 10240
ZROWS = TBL // NS
ZSTEP = 128

@functools.cache
def _mesh():
    return plsc.VectorSubcoreMesh(
        core_axis_name="c", subcore_axis_name="s", num_cores=NC, num_subcores=NS
    )


def _zero_stripe(acc, zbuf, s):
    def zrow(i, _):
        zbuf[i, :] = jnp.zeros((16,), jnp.float32)
        return 0

    lax.fori_loop(0, ZSTEP, zrow, 0)

    def zcp(i, _):
        pltpu.sync_copy(zbuf, acc.at[pl.ds(s * ZROWS + i * ZSTEP, ZSTEP)])
        return 0

    lax.fori_loop(0, ZROWS // ZSTEP, zcp, 0)


def _copy_out(acc, out_hbm, c, s):
    pltpu.sync_copy(
        acc.at[pl.ds(s * ZROWS, ZROWS)], out_hbm.at[c, pl.ds(s * ZROWS, ZROWS)]
    )


@functools.cache
def _sc_degree_kernel():
    return pl.kernel(
        _sc_degree_body,
        out_type=jax.ShapeDtypeStruct((NC, TBL, D_HID), jnp.float32),
        mesh=_mesh(),
        compiler_params=pltpu.CompilerParams(use_tc_tiling_on_sc=False),
        scratch_types=[
            pltpu.VMEM_SHARED((TBL, D_HID), jnp.float32),
            pltpu.VMEM((NCHUNK, CHUNK), jnp.int32),
            pltpu.VMEM((CHUNK, D_HID), jnp.float32),
            pltpu.VMEM((ZSTEP, D_HID), jnp.float32),
            pltpu.SemaphoreType.DMA,
        ],
    )


DEG_WIN = 8


def _sc_degree_body(col_hbm, out_hbm, acc, cidx, ones_b, zbuf, ssem):
    c = lax.axis_index("c")
    s = lax.axis_index("s")
    wid = c * NS + s
    _zero_stripe(acc, zbuf, s)

    def orow(i, _):
        ones_b[i, :] = jnp.ones((16,), jnp.float32)
        return 0

    lax.fori_loop(0, CHUNK, orow, 0)
    pltpu.sync_copy(col_hbm.at[wid], cidx)
    plsc.subcore_barrier()

    def step(j, _):
        @pl.when(j >= DEG_WIN)
        def _():
            pltpu.make_async_copy(ones_b, acc.at[cidx.at[j - DEG_WIN]], ssem).wait()

        pltpu.async_copy(ones_b, acc.at[cidx.at[j]], ssem, add=True)
        return 0

    lax.fori_loop(0, NCHUNK, step, 0)

    def drain(j, _):
        pltpu.make_async_copy(ones_b, acc.at[cidx.at[NCHUNK - DEG_WIN + j]], ssem).wait()
        return 0

    lax.fori_loop(0, DEG_WIN, drain, 0)
    plsc.subcore_barrier()
    _copy_out(acc, out_hbm, c, s)


@functools.cache
def _sc_propagate_kernel():
    return pl.kernel(
        _sc_propagate_body,
        out_type=jax.ShapeDtypeStruct((NC, TBL, D_HID), jnp.float32),
        mesh=_mesh(),
        compiler_params=pltpu.CompilerParams(use_tc_tiling_on_sc=False),
        scratch_types=[
            pltpu.VMEM_SHARED((TBL, D_HID), jnp.float32),
            pltpu.VMEM((NCHUNK, CHUNK), jnp.int32),
            pltpu.VMEM((NCHUNK, CHUNK), jnp.int32),
            pltpu.VMEM((NBUF, CHUNK, D_HID), jnp.float32),
            pltpu.VMEM((ZSTEP, D_HID), jnp.float32),
            pltpu.SemaphoreType.DMA,
            pltpu.SemaphoreType.DMA,
        ],
    )


NBUF = 8
AHEAD = NBUF // 2


def _sc_propagate_body(
    row_hbm, col_hbm, tbl_hbm, out_hbm, acc, ridx, cidx, bufs, zbuf, gsem, ssem
):
    c = lax.axis_index("c")
    s = lax.axis_index("s")
    wid = c * NS + s
    _zero_stripe(acc, zbuf, s)
    pltpu.sync_copy(row_hbm.at[wid], ridx)
    pltpu.sync_copy(col_hbm.at[wid], cidx)
    plsc.subcore_barrier()

    for j in range(AHEAD):
        pltpu.async_copy(tbl_hbm.at[ridx.at[j]], bufs.at[j], gsem)

    def outer(i, _):
        for bb in range(NBUF):
            j = NBUF * i + bb
            pltpu.make_async_copy(tbl_hbm.at[ridx.at[j]], bufs.at[bb], gsem).wait()
            pltpu.async_copy(bufs.at[bb], acc.at[cidx.at[j]], ssem, add=True)

            @pl.when(j >= AHEAD)
            def _():
                pltpu.make_async_copy(
                    bufs.at[(bb + AHEAD) % NBUF], acc.at[cidx.at[j - AHEAD]], ssem
                ).wait()

            @pl.when(j + AHEAD < NCHUNK)
            def _():
                pltpu.async_copy(
                    tbl_hbm.at[ridx.at[j + AHEAD]], bufs.at[(bb + AHEAD) % NBUF], gsem
                )
        return 0

    lax.fori_loop(0, NCHUNK // NBUF, outer, 0)
    for k in range(AHEAD):
        j = NCHUNK - AHEAD + k
        pltpu.make_async_copy(
            bufs.at[j % NBUF], acc.at[cidx.at[j]], ssem
        ).wait()
    plsc.subcore_barrier()
    _copy_out(acc, out_hbm, c, s)


ROWS_B = 2000
GRID = N_NODES // ROWS_B


def _tc_prep_body(x_ref, w_ref, deg_ref, a0_ref, ys_ref, dis_ref):
    deg = deg_ref[0] + deg_ref[1]
    dis = jnp.where(deg > 0, lax.rsqrt(deg), 0.0)
    xw = jnp.dot(x_ref[...], w_ref[...], preferred_element_type=jnp.float32)
    a0_ref[...] = xw[:, :D_HID]
    ys_ref[...] = dis * xw[:, D_HID:]
    dis_ref[...] = dis


def _tc_prep(x, w1c, degp):
    return pl.pallas_call(
        _tc_prep_body,
        grid=(GRID,),
        in_specs=[
            pl.BlockSpec((ROWS_B, D_FEAT), lambda i: (i, 0)),
            pl.BlockSpec((D_FEAT, 2 * D_HID), lambda i: (0, 0)),
            pl.BlockSpec((NC, ROWS_B, D_HID), lambda i: (0, i, 0)),
        ],
        out_specs=[pl.BlockSpec((ROWS_B, D_HID), lambda i: (i, 0))] * 3,
        out_shape=[jax.ShapeDtypeStruct((N_NODES, D_HID), jnp.float32)] * 3,
    )(x, w1c, degp)


def _tc_mid_body(a0_ref, pp_ref, dis_ref, w_ref, b1_ref, c0_ref, zs_ref):
    dis = dis_ref[...]
    h = jnp.maximum(a0_ref[...] + dis * (pp_ref[0] + pp_ref[1]) + b1_ref[...], 0.0)
    hw = jnp.dot(h, w_ref[...], preferred_element_type=jnp.float32)
    c0_ref[...] = hw[:, :D_HID]
    zs_ref[...] = dis * hw[:, D_HID:]


def _tc_mid(a0, pparts, dis, w2c, b1):
    return pl.pallas_call(
        _tc_mid_body,
        grid=(GRID,),
        in_specs=[
            pl.BlockSpec((ROWS_B, D_HID), lambda i: (i, 0)),
            pl.BlockSpec((NC, ROWS_B, D_HID), lambda i: (0, i, 0)),
            pl.BlockSpec((ROWS_B, D_HID), lambda i: (i, 0)),
            pl.BlockSpec((D_HID, 2 * D_HID), lambda i: (0, 0)),
            pl.BlockSpec((1, D_HID), lambda i: (0, 0)),
        ],
        out_specs=[pl.BlockSpec((ROWS_B, D_HID), lambda i: (i, 0))] * 2,
        out_shape=[jax.ShapeDtypeStruct((N_NODES, D_HID), jnp.float32)] * 2,
    )(a0, pparts, dis, w2c, b1)


def _tc_final_body(c0_ref, qp_ref, dis_ref, b2_ref, out_ref):
    o = c0_ref[...] + dis_ref[...] * (qp_ref[0] + qp_ref[1]) + b2_ref[...]
    m = jnp.max(o, axis=1, keepdims=True)
    e = jnp.exp(o - m)
    out_ref[...] = (o - m) - jnp.log(jnp.sum(e, axis=1, keepdims=True))


def _tc_final(c0, qparts, dis, b2):
    return pl.pallas_call(
        _tc_final_body,
        grid=(GRID,),
        in_specs=[
            pl.BlockSpec((ROWS_B, D_HID), lambda i: (i, 0)),
            pl.BlockSpec((NC, ROWS_B, D_HID), lambda i: (0, i, 0)),
            pl.BlockSpec((ROWS_B, D_HID), lambda i: (i, 0)),
            pl.BlockSpec((1, D_HID), lambda i: (0, 0)),
        ],
        out_specs=pl.BlockSpec((ROWS_B, D_HID), lambda i: (i, 0)),
        out_shape=jax.ShapeDtypeStruct((N_NODES, D_HID), jnp.float32),
    )(c0, qparts, dis, b2)


def kernel(x, edge_index, W1_0, W1_1, b1, W2_0, W2_1, b2):
    ei = edge_index.astype(jnp.int32)
    row3 = ei[0].reshape(NW, NCHUNK, CHUNK)
    col3 = ei[1].reshape(NW, NCHUNK, CHUNK)
    degp = _sc_degree_kernel()(col3)
    w1c = jnp.concatenate([W1_0, W1_1], axis=1)
    a0, ys, dis = _tc_prep(x, w1c, degp)
    pparts = _sc_propagate_kernel()(row3, col3, ys)
    w2c = jnp.concatenate([W2_0, W2_1], axis=1)
    c0, zs = _tc_mid(a0, pparts, dis, w2c, b1.reshape(1, D_HID))
    qparts = _sc_propagate_kernel()(row3, col3, zs)
    return _tc_final(c0, qparts, dis, b2.reshape(1, D_HID))


# speedup vs baseline: 51.6081x; 1.1397x over previous
import functools

import jax
import jax.numpy as jnp
from jax import lax
from jax.experimental import pallas as pl
from jax.experimental.pallas import tpu as pltpu
from jax.experimental.pallas import tpu_sc as plsc

N_NODES = 10000
N_EDGES = 320000
D_FEAT = 128
D_HID = 16

NC = 2
NS = 16
NW = NC * NS
EPT = N_EDGES // NW
CHUNK = 125
NCHUNK = EPT // CHUNK
TBL = 10240
ZROWS = TBL // NS
ZSTEP = 128

@functools.cache
def _mesh():
    return plsc.VectorSubcoreMesh(
        core_axis_name="c", subcore_axis_name="s", num_cores=NC, num_subcores=NS
    )


def _zero_stripe(acc, zbuf, s):
    def zrow(i, _):
        zbuf[i, :] = jnp.zeros((16,), jnp.float32)
        return 0

    lax.fori_loop(0, ZSTEP, zrow, 0)

    def zcp(i, _):
        pltpu.sync_copy(zbuf, acc.at[pl.ds(s * ZROWS + i * ZSTEP, ZSTEP)])
        return 0

    lax.fori_loop(0, ZROWS // ZSTEP, zcp, 0)


def _copy_out(acc, out_hbm, c, s):
    pltpu.sync_copy(
        acc.at[pl.ds(s * ZROWS, ZROWS)], out_hbm.at[c, pl.ds(s * ZROWS, ZROWS)]
    )


@functools.cache
def _sc_degree_kernel():
    return pl.kernel(
        _sc_degree_body,
        out_type=jax.ShapeDtypeStruct((NC, TBL, D_HID), jnp.float32),
        mesh=_mesh(),
        compiler_params=pltpu.CompilerParams(use_tc_tiling_on_sc=False),
        scratch_types=[
            pltpu.VMEM_SHARED((TBL, D_HID), jnp.float32),
            pltpu.VMEM((NCHUNK, CHUNK), jnp.int32),
            pltpu.VMEM((CHUNK, D_HID), jnp.float32),
            pltpu.VMEM((ZSTEP, D_HID), jnp.float32),
            pltpu.SemaphoreType.DMA,
        ],
    )


DEG_WIN = 8


def _sc_degree_body(col_hbm, out_hbm, acc, cidx, ones_b, zbuf, ssem):
    c = lax.axis_index("c")
    s = lax.axis_index("s")
    wid = c * NS + s
    _zero_stripe(acc, zbuf, s)

    def orow(i, _):
        ones_b[i, :] = jnp.ones((16,), jnp.float32)
        return 0

    lax.fori_loop(0, CHUNK, orow, 0)
    pltpu.sync_copy(col_hbm.at[wid], cidx)
    plsc.subcore_barrier()

    def step(j, _):
        @pl.when(j >= DEG_WIN)
        def _():
            pltpu.make_async_copy(ones_b, acc.at[cidx.at[j - DEG_WIN]], ssem).wait()

        pltpu.async_copy(ones_b, acc.at[cidx.at[j]], ssem, add=True)
        return 0

    lax.fori_loop(0, NCHUNK, step, 0)

    def drain(j, _):
        pltpu.make_async_copy(ones_b, acc.at[cidx.at[NCHUNK - DEG_WIN + j]], ssem).wait()
        return 0

    lax.fori_loop(0, DEG_WIN, drain, 0)
    plsc.subcore_barrier()
    _copy_out(acc, out_hbm, c, s)


@functools.cache
def _sc_propagate_kernel():
    return pl.kernel(
        _sc_propagate_body,
        out_type=jax.ShapeDtypeStruct((NC, TBL, D_HID), jnp.float32),
        mesh=_mesh(),
        compiler_params=pltpu.CompilerParams(use_tc_tiling_on_sc=False),
        scratch_types=[
            pltpu.VMEM_SHARED((TBL, D_HID), jnp.float32),
            pltpu.VMEM((NCHUNK, CHUNK), jnp.int32),
            pltpu.VMEM((NCHUNK, CHUNK), jnp.int32),
            pltpu.VMEM((NBUF, CHUNK, D_HID), jnp.float32),
            pltpu.VMEM((ZSTEP, D_HID), jnp.float32),
            pltpu.SemaphoreType.DMA,
            pltpu.SemaphoreType.DMA,
        ],
    )


NBUF = 16
AHEAD = NBUF // 2


def _sc_propagate_body(
    row_hbm, col_hbm, tbl_hbm, out_hbm, acc, ridx, cidx, bufs, zbuf, gsem, ssem
):
    c = lax.axis_index("c")
    s = lax.axis_index("s")
    wid = c * NS + s
    _zero_stripe(acc, zbuf, s)
    pltpu.sync_copy(row_hbm.at[wid], ridx)
    pltpu.sync_copy(col_hbm.at[wid], cidx)
    plsc.subcore_barrier()

    for j in range(AHEAD):
        pltpu.async_copy(tbl_hbm.at[ridx.at[j]], bufs.at[j], gsem)

    def outer(i, _):
        for bb in range(NBUF):
            j = NBUF * i + bb
            pltpu.make_async_copy(tbl_hbm.at[ridx.at[j]], bufs.at[bb], gsem).wait()
            pltpu.async_copy(bufs.at[bb], acc.at[cidx.at[j]], ssem, add=True)

            @pl.when(j >= AHEAD)
            def _():
                pltpu.make_async_copy(
                    bufs.at[(bb + AHEAD) % NBUF], acc.at[cidx.at[j - AHEAD]], ssem
                ).wait()

            @pl.when(j + AHEAD < NCHUNK)
            def _():
                pltpu.async_copy(
                    tbl_hbm.at[ridx.at[j + AHEAD]], bufs.at[(bb + AHEAD) % NBUF], gsem
                )
        return 0

    lax.fori_loop(0, NCHUNK // NBUF, outer, 0)
    for k in range(AHEAD):
        j = NCHUNK - AHEAD + k
        pltpu.make_async_copy(
            bufs.at[j % NBUF], acc.at[cidx.at[j]], ssem
        ).wait()
    plsc.subcore_barrier()
    _copy_out(acc, out_hbm, c, s)


ROWS_B = 2000
GRID = N_NODES // ROWS_B


def _tc_prep_body(x_ref, w_ref, deg_ref, a0_ref, ys_ref, dis_ref):
    deg = deg_ref[0] + deg_ref[1]
    dis = jnp.where(deg > 0, lax.rsqrt(deg), 0.0)
    xw = jnp.dot(x_ref[...], w_ref[...], preferred_element_type=jnp.float32)
    a0_ref[...] = xw[:, :D_HID]
    ys_ref[...] = dis * xw[:, D_HID:]
    dis_ref[...] = dis


def _tc_prep(x, w1c, degp):
    return pl.pallas_call(
        _tc_prep_body,
        grid=(GRID,),
        in_specs=[
            pl.BlockSpec((ROWS_B, D_FEAT), lambda i: (i, 0)),
            pl.BlockSpec((D_FEAT, 2 * D_HID), lambda i: (0, 0)),
            pl.BlockSpec((NC, ROWS_B, D_HID), lambda i: (0, i, 0)),
        ],
        out_specs=[pl.BlockSpec((ROWS_B, D_HID), lambda i: (i, 0))] * 3,
        out_shape=[jax.ShapeDtypeStruct((N_NODES, D_HID), jnp.float32)] * 3,
    )(x, w1c, degp)


def _tc_mid_body(a0_ref, pp_ref, dis_ref, w_ref, b1_ref, c0_ref, zs_ref):
    dis = dis_ref[...]
    h = jnp.maximum(a0_ref[...] + dis * (pp_ref[0] + pp_ref[1]) + b1_ref[...], 0.0)
    hw = jnp.dot(h, w_ref[...], preferred_element_type=jnp.float32)
    c0_ref[...] = hw[:, :D_HID]
    zs_ref[...] = dis * hw[:, D_HID:]


def _tc_mid(a0, pparts, dis, w2c, b1):
    return pl.pallas_call(
        _tc_mid_body,
        grid=(GRID,),
        in_specs=[
            pl.BlockSpec((ROWS_B, D_HID), lambda i: (i, 0)),
            pl.BlockSpec((NC, ROWS_B, D_HID), lambda i: (0, i, 0)),
            pl.BlockSpec((ROWS_B, D_HID), lambda i: (i, 0)),
            pl.BlockSpec((D_HID, 2 * D_HID), lambda i: (0, 0)),
            pl.BlockSpec((1, D_HID), lambda i: (0, 0)),
        ],
        out_specs=[pl.BlockSpec((ROWS_B, D_HID), lambda i: (i, 0))] * 2,
        out_shape=[jax.ShapeDtypeStruct((N_NODES, D_HID), jnp.float32)] * 2,
    )(a0, pparts, dis, w2c, b1)


def _tc_final_body(c0_ref, qp_ref, dis_ref, b2_ref, out_ref):
    o = c0_ref[...] + dis_ref[...] * (qp_ref[0] + qp_ref[1]) + b2_ref[...]
    m = jnp.max(o, axis=1, keepdims=True)
    e = jnp.exp(o - m)
    out_ref[...] = (o - m) - jnp.log(jnp.sum(e, axis=1, keepdims=True))


def _tc_final(c0, qparts, dis, b2):
    return pl.pallas_call(
        _tc_final_body,
        grid=(GRID,),
        in_specs=[
            pl.BlockSpec((ROWS_B, D_HID), lambda i: (i, 0)),
            pl.BlockSpec((NC, ROWS_B, D_HID), lambda i: (0, i, 0)),
            pl.BlockSpec((ROWS_B, D_HID), lambda i: (i, 0)),
            pl.BlockSpec((1, D_HID), lambda i: (0, 0)),
        ],
        out_specs=pl.BlockSpec((ROWS_B, D_HID), lambda i: (i, 0)),
        out_shape=jax.ShapeDtypeStruct((N_NODES, D_HID), jnp.float32),
    )(c0, qparts, dis, b2)


def kernel(x, edge_index, W1_0, W1_1, b1, W2_0, W2_1, b2):
    ei = edge_index.astype(jnp.int32)
    row3 = ei[0].reshape(NW, NCHUNK, CHUNK)
    col3 = ei[1].reshape(NW, NCHUNK, CHUNK)
    degp = _sc_degree_kernel()(col3)
    w1c = jnp.concatenate([W1_0, W1_1], axis=1)
    a0, ys, dis = _tc_prep(x, w1c, degp)
    pparts = _sc_propagate_kernel()(row3, col3, ys)
    w2c = jnp.concatenate([W2_0, W2_1], axis=1)
    c0, zs = _tc_mid(a0, pparts, dis, w2c, b1.reshape(1, D_HID))
    qparts = _sc_propagate_kernel()(row3, col3, zs)
    return _tc_final(c0, qparts, dis, b2.reshape(1, D_HID))


# speedup vs baseline: 53.4207x; 1.0351x over previous
import functools

import jax
import jax.numpy as jnp
from jax import lax
from jax.experimental import pallas as pl
from jax.experimental.pallas import tpu as pltpu
from jax.experimental.pallas import tpu_sc as plsc

N_NODES = 10000
N_EDGES = 320000
D_FEAT = 128
D_HID = 16

NC = 2
NS = 16
NW = NC * NS
EPT = N_EDGES // NW
CHUNK = 125
NCHUNK = EPT // CHUNK
TBL = 10240
ZROWS = TBL // NS
ZSTEP = 128

@functools.cache
def _mesh():
    return plsc.VectorSubcoreMesh(
        core_axis_name="c", subcore_axis_name="s", num_cores=NC, num_subcores=NS
    )


def _zero_stripe(acc, zbuf, s):
    def zrow(i, _):
        zbuf[i, :] = jnp.zeros((16,), jnp.float32)
        return 0

    lax.fori_loop(0, ZSTEP, zrow, 0)

    def zcp(i, _):
        pltpu.sync_copy(zbuf, acc.at[pl.ds(s * ZROWS + i * ZSTEP, ZSTEP)])
        return 0

    lax.fori_loop(0, ZROWS // ZSTEP, zcp, 0)


def _copy_out(acc, out_hbm, c, s):
    pltpu.sync_copy(
        acc.at[pl.ds(s * ZROWS, ZROWS)], out_hbm.at[c, pl.ds(s * ZROWS, ZROWS)]
    )


@functools.cache
def _sc_degree_kernel():
    return pl.kernel(
        _sc_degree_body,
        out_type=jax.ShapeDtypeStruct((NC, TBL, D_HID), jnp.float32),
        mesh=_mesh(),
        compiler_params=pltpu.CompilerParams(use_tc_tiling_on_sc=False),
        scratch_types=[
            pltpu.VMEM_SHARED((TBL, D_HID), jnp.float32),
            pltpu.VMEM((NCHUNK, CHUNK), jnp.int32),
            pltpu.VMEM((CHUNK, D_HID), jnp.float32),
            pltpu.VMEM((ZSTEP, D_HID), jnp.float32),
            pltpu.SemaphoreType.DMA,
        ],
    )


DEG_WIN = 8


def _sc_degree_body(col_hbm, out_hbm, acc, cidx, ones_b, zbuf, ssem):
    c = lax.axis_index("c")
    s = lax.axis_index("s")
    wid = c * NS + s
    _zero_stripe(acc, zbuf, s)

    def orow(i, _):
        ones_b[i, :] = jnp.ones((16,), jnp.float32)
        return 0

    lax.fori_loop(0, CHUNK, orow, 0)
    pltpu.sync_copy(col_hbm.at[wid], cidx)
    plsc.subcore_barrier()

    def step(j, _):
        @pl.when(j >= DEG_WIN)
        def _():
            pltpu.make_async_copy(ones_b, acc.at[cidx.at[j - DEG_WIN]], ssem).wait()

        pltpu.async_copy(ones_b, acc.at[cidx.at[j]], ssem, add=True)
        return 0

    lax.fori_loop(0, NCHUNK, step, 0)

    def drain(j, _):
        pltpu.make_async_copy(ones_b, acc.at[cidx.at[NCHUNK - DEG_WIN + j]], ssem).wait()
        return 0

    lax.fori_loop(0, DEG_WIN, drain, 0)
    plsc.subcore_barrier()
    _copy_out(acc, out_hbm, c, s)


@functools.cache
def _sc_propagate_kernel():
    return pl.kernel(
        _sc_propagate_body,
        out_type=jax.ShapeDtypeStruct((NC, TBL, D_HID), jnp.float32),
        mesh=_mesh(),
        compiler_params=pltpu.CompilerParams(use_tc_tiling_on_sc=False),
        scratch_types=[
            pltpu.VMEM_SHARED((TBL, D_HID), jnp.float32),
            pltpu.VMEM((NCHUNK, CHUNK), jnp.int32),
            pltpu.VMEM((NCHUNK, CHUNK), jnp.int32),
            pltpu.VMEM((NBUF, CHUNK, D_HID), jnp.float32),
            pltpu.VMEM((ZSTEP, D_HID), jnp.float32),
            pltpu.SemaphoreType.DMA,
            pltpu.SemaphoreType.DMA,
        ],
    )


NBUF = 40
AHEAD = NBUF // 2


def _sc_propagate_body(
    row_hbm, col_hbm, tbl_hbm, out_hbm, acc, ridx, cidx, bufs, zbuf, gsem, ssem
):
    c = lax.axis_index("c")
    s = lax.axis_index("s")
    wid = c * NS + s
    _zero_stripe(acc, zbuf, s)
    pltpu.sync_copy(row_hbm.at[wid], ridx)
    pltpu.sync_copy(col_hbm.at[wid], cidx)
    plsc.subcore_barrier()

    for j in range(AHEAD):
        pltpu.async_copy(tbl_hbm.at[ridx.at[j]], bufs.at[j], gsem)

    def outer(i, _):
        for bb in range(NBUF):
            j = NBUF * i + bb
            pltpu.make_async_copy(tbl_hbm.at[ridx.at[j]], bufs.at[bb], gsem).wait()
            pltpu.async_copy(bufs.at[bb], acc.at[cidx.at[j]], ssem, add=True)

            @pl.when(j >= AHEAD)
            def _():
                pltpu.make_async_copy(
                    bufs.at[(bb + AHEAD) % NBUF], acc.at[cidx.at[j - AHEAD]], ssem
                ).wait()

            @pl.when(j + AHEAD < NCHUNK)
            def _():
                pltpu.async_copy(
                    tbl_hbm.at[ridx.at[j + AHEAD]], bufs.at[(bb + AHEAD) % NBUF], gsem
                )
        return 0

    lax.fori_loop(0, NCHUNK // NBUF, outer, 0)
    for k in range(AHEAD):
        j = NCHUNK - AHEAD + k
        pltpu.make_async_copy(
            bufs.at[j % NBUF], acc.at[cidx.at[j]], ssem
        ).wait()
    plsc.subcore_barrier()
    _copy_out(acc, out_hbm, c, s)


ROWS_B = 2000
GRID = N_NODES // ROWS_B


def _tc_prep_body(x_ref, w_ref, deg_ref, a0_ref, ys_ref, dis_ref):
    deg = deg_ref[0] + deg_ref[1]
    dis = jnp.where(deg > 0, lax.rsqrt(deg), 0.0)
    xw = jnp.dot(x_ref[...], w_ref[...], preferred_element_type=jnp.float32)
    a0_ref[...] = xw[:, :D_HID]
    ys_ref[...] = dis * xw[:, D_HID:]
    dis_ref[...] = dis


def _tc_prep(x, w1c, degp):
    return pl.pallas_call(
        _tc_prep_body,
        grid=(GRID,),
        in_specs=[
            pl.BlockSpec((ROWS_B, D_FEAT), lambda i: (i, 0)),
            pl.BlockSpec((D_FEAT, 2 * D_HID), lambda i: (0, 0)),
            pl.BlockSpec((NC, ROWS_B, D_HID), lambda i: (0, i, 0)),
        ],
        out_specs=[pl.BlockSpec((ROWS_B, D_HID), lambda i: (i, 0))] * 3,
        out_shape=[jax.ShapeDtypeStruct((N_NODES, D_HID), jnp.float32)] * 3,
    )(x, w1c, degp)


def _tc_mid_body(a0_ref, pp_ref, dis_ref, w_ref, b1_ref, c0_ref, zs_ref):
    dis = dis_ref[...]
    h = jnp.maximum(a0_ref[...] + dis * (pp_ref[0] + pp_ref[1]) + b1_ref[...], 0.0)
    hw = jnp.dot(h, w_ref[...], preferred_element_type=jnp.float32)
    c0_ref[...] = hw[:, :D_HID]
    zs_ref[...] = dis * hw[:, D_HID:]


def _tc_mid(a0, pparts, dis, w2c, b1):
    return pl.pallas_call(
        _tc_mid_body,
        grid=(GRID,),
        in_specs=[
            pl.BlockSpec((ROWS_B, D_HID), lambda i: (i, 0)),
            pl.BlockSpec((NC, ROWS_B, D_HID), lambda i: (0, i, 0)),
            pl.BlockSpec((ROWS_B, D_HID), lambda i: (i, 0)),
            pl.BlockSpec((D_HID, 2 * D_HID), lambda i: (0, 0)),
            pl.BlockSpec((1, D_HID), lambda i: (0, 0)),
        ],
        out_specs=[pl.BlockSpec((ROWS_B, D_HID), lambda i: (i, 0))] * 2,
        out_shape=[jax.ShapeDtypeStruct((N_NODES, D_HID), jnp.float32)] * 2,
    )(a0, pparts, dis, w2c, b1)


def _tc_final_body(c0_ref, qp_ref, dis_ref, b2_ref, out_ref):
    o = c0_ref[...] + dis_ref[...] * (qp_ref[0] + qp_ref[1]) + b2_ref[...]
    m = jnp.max(o, axis=1, keepdims=True)
    e = jnp.exp(o - m)
    out_ref[...] = (o - m) - jnp.log(jnp.sum(e, axis=1, keepdims=True))


def _tc_final(c0, qparts, dis, b2):
    return pl.pallas_call(
        _tc_final_body,
        grid=(GRID,),
        in_specs=[
            pl.BlockSpec((ROWS_B, D_HID), lambda i: (i, 0)),
            pl.BlockSpec((NC, ROWS_B, D_HID), lambda i: (0, i, 0)),
            pl.BlockSpec((ROWS_B, D_HID), lambda i: (i, 0)),
            pl.BlockSpec((1, D_HID), lambda i: (0, 0)),
        ],
        out_specs=pl.BlockSpec((ROWS_B, D_HID), lambda i: (i, 0)),
        out_shape=jax.ShapeDtypeStruct((N_NODES, D_HID), jnp.float32),
    )(c0, qparts, dis, b2)


def kernel(x, edge_index, W1_0, W1_1, b1, W2_0, W2_1, b2):
    ei = edge_index.astype(jnp.int32)
    row3 = ei[0].reshape(NW, NCHUNK, CHUNK)
    col3 = ei[1].reshape(NW, NCHUNK, CHUNK)
    degp = _sc_degree_kernel()(col3)
    w1c = jnp.concatenate([W1_0, W1_1], axis=1)
    a0, ys, dis = _tc_prep(x, w1c, degp)
    pparts = _sc_propagate_kernel()(row3, col3, ys)
    w2c = jnp.concatenate([W2_0, W2_1], axis=1)
    c0, zs = _tc_mid(a0, pparts, dis, w2c, b1.reshape(1, D_HID))
    qparts = _sc_propagate_kernel()(row3, col3, zs)
    return _tc_final(c0, qparts, dis, b2.reshape(1, D_HID))
